# baseline (speedup 1.0000x reference)
import functools

import jax
import jax.numpy as jnp
from jax import lax
from jax.experimental import pallas as pl
from jax.experimental.pallas import tpu as pltpu
from jax.experimental.pallas import tpu_sc as plsc

D_MODEL = 64
SCALE = float(D_MODEL) ** 0.5
L = 16


@functools.lru_cache(maxsize=None)
def _build_embed(B: int, V: int):
    info = plsc.get_sparse_core_info()
    NC, NS = info.num_cores, info.num_subcores
    NW = NC * NS
    G = 128
    NG = 4
    C = NG * G
    assert B % (NW * C) == 0
    b_per_w = B // NW
    n_chunks = b_per_w // C
    xrows_per_w = b_per_w // G

    mesh = plsc.VectorSubcoreMesh(core_axis_name="c", subcore_axis_name="s")

    @functools.partial(
        pl.kernel,
        out_type=jax.ShapeDtypeStruct((B, D_MODEL), jnp.float32),
        mesh=mesh,
        scratch_types=[
            pltpu.VMEM((NG, G), jnp.int32),
            pltpu.VMEM((C, D_MODEL), jnp.float32),
            pltpu.SemaphoreType.DMA,
        ],
        compiler_params=pltpu.CompilerParams(use_tc_tiling_on_sc=False),
    )
    def embed(x_hbm, table_hbm, out_hbm, idx_v, rows_v, gsem):
        wid = lax.axis_index("s") * NC + lax.axis_index("c")

        def chunk_fn(c, carry):
            xrow0 = wid * xrows_per_w + c * NG
            pltpu.sync_copy(x_hbm.at[pl.ds(xrow0, NG)], idx_v)
            copies = [
                pltpu.async_copy(
                    table_hbm.at[idx_v.at[j]],
                    rows_v.at[pl.ds(j * G, G)],
                    gsem,
                )
                for j in range(NG)
            ]
            for cp in copies:
                cp.wait()

            def row_fn(i, rcarry):
                for j4 in range(D_MODEL // L):
                    rows_v[i, pl.ds(j4 * L, L)] = (
                        rows_v[i, pl.ds(j4 * L, L)] * SCALE
                    )
                return rcarry

            lax.fori_loop(0, C, row_fn, 0, unroll=2)
            pltpu.sync_copy(
                rows_v, out_hbm.at[pl.ds(wid * b_per_w + c * C, C)]
            )
            return carry

        lax.fori_loop(0, n_chunks, chunk_fn, 0)

    return embed


def kernel(x, table):
    s1, s2 = x.shape
    B = s1 * s2
    V, d = table.shape
    xf = x.reshape(B // 128, 128).astype(jnp.int32)
    out = _build_embed(B, V)(xf, table)
    return out.reshape(s1, s2, d)


# speedup vs baseline: 1.0886x; 1.0886x over previous
import functools

import jax
import jax.numpy as jnp
from jax import lax
from jax.experimental import pallas as pl
from jax.experimental.pallas import tpu as pltpu
from jax.experimental.pallas import tpu_sc as plsc

D_MODEL = 64
SCALE = float(D_MODEL) ** 0.5
L = 16
G = 128
NG = 2
C = NG * G
NBUF = 4


@functools.lru_cache(maxsize=None)
def _build_embed(B: int, V: int):
    info = plsc.get_sparse_core_info()
    NC, NS = info.num_cores, info.num_subcores
    NW = NC * NS
    assert B % (NW * C) == 0
    b_per_w = B // NW
    n_chunks = b_per_w // C
    ir_per_w = b_per_w // G
    assert n_chunks % NBUF == 0 and n_chunks >= 2 * NBUF

    mesh = plsc.VectorSubcoreMesh(core_axis_name="c", subcore_axis_name="s")

    @functools.partial(
        pl.kernel,
        out_type=jax.ShapeDtypeStruct((B, D_MODEL), jnp.float32),
        mesh=mesh,
        scratch_types=[
            pltpu.VMEM((ir_per_w, G), jnp.int32),
            pltpu.VMEM((NBUF, C, D_MODEL), jnp.float32),
        ]
        + [pltpu.SemaphoreType.DMA] * (2 * NBUF),
        compiler_params=pltpu.CompilerParams(use_tc_tiling_on_sc=False),
    )
    def embed(x_hbm, table_hbm, out_hbm, idx_all, rows_v, *sems):
        gsems, osems = sems[:NBUF], sems[NBUF:]
        wid = lax.axis_index("s") * NC + lax.axis_index("c")
        out_base = wid * b_per_w
        pltpu.sync_copy(x_hbm.at[pl.ds(wid * ir_per_w, ir_per_w)], idx_all)

        def fire_gathers(c, s):
            for j in range(NG):
                pltpu.async_copy(
                    table_hbm.at[idx_all.at[c * NG + j]],
                    rows_v.at[s, pl.ds(j * G, G)],
                    gsems[s],
                )

        def drain_gathers(c, s):
            for j in range(NG):
                pltpu.make_async_copy(
                    table_hbm.at[idx_all.at[c * NG + j]],
                    rows_v.at[s, pl.ds(j * G, G)],
                    gsems[s],
                ).wait()

        def fire_out(c, s):
            pltpu.async_copy(
                rows_v.at[s], out_hbm.at[pl.ds(out_base + c * C, C)], osems[s]
            )

        def wait_out(c, s):
            pltpu.make_async_copy(
                rows_v.at[s], out_hbm.at[pl.ds(out_base + c * C, C)], osems[s]
            ).wait()

        def scale(s):
            def row_fn(i, carry):
                for j4 in range(D_MODEL // L):
                    rows_v[s, i, pl.ds(j4 * L, L)] = (
                        rows_v[s, i, pl.ds(j4 * L, L)] * SCALE
                    )
                return carry

            lax.fori_loop(0, C, row_fn, 0, unroll=4)

        fire_gathers(0, 0)
        fire_gathers(1, 1)

        def step(c0, carry):
            for k in range(NBUF):
                c = c0 * NBUF + k
                s = k
                s2 = (k + 2) % NBUF

                @pl.when(c >= 2)
                def _():
                    wait_out(c - 2, s2)

                @pl.when(c + 2 < n_chunks)
                def _():
                    fire_gathers(c + 2, s2)

                drain_gathers(c, s)
                scale(s)
                fire_out(c, s)
            return carry

        lax.fori_loop(0, n_chunks // NBUF, step, 0)
        wait_out(n_chunks - 2, (n_chunks - 2) % NBUF)
        wait_out(n_chunks - 1, (n_chunks - 1) % NBUF)

    return embed


def kernel(x, table):
    s1, s2 = x.shape
    B = s1 * s2
    V, d = table.shape
    xf = x.reshape(B // G, G).astype(jnp.int32)
    out = _build_embed(B, V)(xf, table)
    return out.reshape(s1, s2, d)


# speedup vs baseline: 1.3315x; 1.2232x over previous
import functools

import jax
import jax.numpy as jnp
from jax import lax
from jax.experimental import pallas as pl
from jax.experimental.pallas import tpu as pltpu
from jax.experimental.pallas import tpu_sc as plsc

D_MODEL = 64
SCALE = float(D_MODEL) ** 0.5
L = 16
G = 128
NBUF = 4


@functools.lru_cache(maxsize=None)
def _build_embed(B: int, V: int):
    info = plsc.get_sparse_core_info()
    NC, NS = info.num_cores, info.num_subcores
    NW = NC * NS
    assert B % (NW * G) == 0
    b_per_w = B // NW
    n_chunks = b_per_w // G
    assert n_chunks % NBUF == 0 and n_chunks >= 2 * NBUF

    mesh = plsc.VectorSubcoreMesh(core_axis_name="c", subcore_axis_name="s")

    @functools.partial(
        pl.kernel,
        out_type=jax.ShapeDtypeStruct((B, 2 * D_MODEL), jnp.float32),
        mesh=mesh,
        scratch_types=[
            pltpu.VMEM((n_chunks, G), jnp.int32),
            pltpu.VMEM((NBUF, G, 2 * D_MODEL), jnp.float32),
        ]
        + [pltpu.SemaphoreType.DMA] * (2 * NBUF),
        compiler_params=pltpu.CompilerParams(use_tc_tiling_on_sc=True),
    )
    def embed(x_hbm, table_hbm, out_hbm, idx_all, rows_v, *sems):
        gsems, osems = sems[:NBUF], sems[NBUF:]
        wid = lax.axis_index("s") * NC + lax.axis_index("c")
        out_base = wid * b_per_w
        pltpu.sync_copy(x_hbm.at[pl.ds(wid * n_chunks, n_chunks)], idx_all)

        def fire_gather(c, s):
            pltpu.async_copy(
                table_hbm.at[idx_all.at[c]], rows_v.at[s], gsems[s]
            )

        def drain_gather(c, s):
            pltpu.make_async_copy(
                table_hbm.at[idx_all.at[c]], rows_v.at[s], gsems[s]
            ).wait()

        def fire_out(c, s):
            pltpu.async_copy(
                rows_v.at[s],
                out_hbm.at[pl.ds(out_base + c * G, G)],
                osems[s],
            )

        def wait_out(c, s):
            pltpu.make_async_copy(
                rows_v.at[s],
                out_hbm.at[pl.ds(out_base + c * G, G)],
                osems[s],
            ).wait()

        def scale(s):
            def row_fn(i, carry):
                for j4 in range(D_MODEL // L):
                    rows_v[s, i, pl.ds(j4 * L, L)] = (
                        rows_v[s, i, pl.ds(j4 * L, L)] * SCALE
                    )
                return carry

            lax.fori_loop(0, G, row_fn, 0, unroll=4)

        fire_gather(0, 0)
        fire_gather(1, 1)

        def step(c0, carry):
            for k in range(NBUF):
                c = c0 * NBUF + k
                s = k
                s2 = (k + 2) % NBUF

                @pl.when(c >= 2)
                def _():
                    wait_out(c - 2, s2)

                @pl.when(c + 2 < n_chunks)
                def _():
                    fire_gather(c + 2, s2)

                drain_gather(c, s)
                scale(s)
                fire_out(c, s)
            return carry

        lax.fori_loop(0, n_chunks // NBUF, step, 0)
        wait_out(n_chunks - 2, (n_chunks - 2) % NBUF)
        wait_out(n_chunks - 1, (n_chunks - 1) % NBUF)

    return embed


def kernel(x, table):
    s1, s2 = x.shape
    B = s1 * s2
    V, d = table.shape
    xf = x.reshape(B // G, G).astype(jnp.int32)
    tpad = jnp.pad(table, ((0, 0), (0, 2 * D_MODEL - d)))
    out = _build_embed(B, V)(xf, tpad)
    return out.reshape(s1, s2, 2 * D_MODEL)[:, :, :d]
